# Initial kernel scaffold; baseline (speedup 1.0000x reference)
import jax
import jax.numpy as jnp
from jax.experimental import pallas as pl


def kernel(res1, fea1, label_bs, gt, queues):
    raise NotImplementedError("write your pallas kernel here")


# speedup vs baseline: 4.4079x; 4.4079x over previous
import functools

import jax
import jax.numpy as jnp
from jax.experimental import pallas as pl
from jax.experimental.pallas import tpu as pltpu

_NUM_CLASSES = 5
_INNER = 256
_TEMP = 0.2
_QLEN = 64
_NC_PAD = 8


def _body(labelbs_ref, fea_ref, res_ref, gt_ref, q_ref, out_ref,
          acc_ref, cnt_ref, tot_ref, *, nblk, bs):
    ii = pl.program_id(0)
    j = pl.program_id(1)

    @pl.when(j == 0)
    def _init():
        acc_ref[...] = jnp.zeros_like(acc_ref)
        cnt_ref[...] = jnp.zeros_like(cnt_ref)

    @pl.when((j == 0) & (ii == 0))
    def _init_total():
        tot_ref[...] = jnp.zeros_like(tot_ref)

    fea = fea_ref[0]
    res = res_ref[0]
    gt_row = gt_ref[0]

    best_val = res[0:1, :]
    best_idx = jnp.zeros_like(gt_row)
    for c in range(1, _NUM_CLASSES):
        row = res[c:c + 1, :]
        upd = row > best_val
        best_val = jnp.where(upd, row, best_val)
        best_idx = jnp.where(upd, jnp.full_like(best_idx, c), best_idx)

    labels = jnp.where(ii < labelbs_ref[0, 0], gt_row, best_idx)

    cls_iota = jax.lax.broadcasted_iota(jnp.int32, (_NC_PAD, labels.shape[1]), 0)
    onehot = (cls_iota == labels).astype(jnp.float32)

    acc_ref[...] += jax.lax.dot_general(
        fea, onehot, (((1,), (1,)), ((), ())),
        preferred_element_type=jnp.float32)
    ones_row = jnp.ones((1, labels.shape[1]), jnp.float32)
    cnt_ref[...] += jax.lax.dot_general(
        ones_row, onehot, (((1,), (1,)), ((), ())),
        preferred_element_type=jnp.float32)

    @pl.when(j == nblk - 1)
    def _sample_loss():
        sums = acc_ref[:, :_NUM_CLASSES]
        counts = cnt_ref[:, :_NUM_CLASSES]
        means = sums / counts
        norm = jnp.sqrt(jnp.sum(means * means, axis=0, keepdims=True))
        keys = means / jnp.maximum(norm, 1e-12)
        logits = jax.lax.dot_general(
            keys, q_ref[...], (((0,), (0,)), ((), ())),
            preferred_element_type=jnp.float32)
        scaled = logits * (1.0 / _TEMP)
        expx = jnp.exp(scaled)
        loss_s = jnp.zeros((), jnp.float32)
        for cls in range(1, _NUM_CLASSES):
            row = scaled[cls - 1:cls, :]
            erow = expx[cls - 1:cls, :]
            l_pos = row[:, cls * _QLEN:(cls + 1) * _QLEN]
            e_pos = erow[:, cls * _QLEN:(cls + 1) * _QLEN]
            neg_base = jnp.sum(erow) - jnp.sum(e_pos)
            log_prob = l_pos - jnp.log(e_pos + neg_base)
            loss_s = loss_s + (-jnp.mean(log_prob))
        tot_ref[...] = tot_ref[...] + loss_s / (_NUM_CLASSES - 1)

    @pl.when((j == nblk - 1) & (ii == bs - 1))
    def _emit():
        out_ref[...] = tot_ref[...] / bs


@jax.jit
def kernel(res1, fea1, label_bs, gt, queues):
    bs, nc, h, w = res1.shape
    inner = fea1.shape[1]
    p = h * w
    pb = 2048
    nblk = p // pb

    fea_v = fea1.reshape(bs, inner, p)
    res_v = res1.reshape(bs, nc, p)
    gt_v = gt.reshape(bs, 1, p)
    q2d = jnp.transpose(queues, (1, 0, 2)).reshape(inner, nc * _QLEN)
    lbs = jnp.asarray(label_bs, jnp.int32).reshape(1, 1)

    grid = (bs, nblk)
    out = pl.pallas_call(
        functools.partial(_body, nblk=nblk, bs=bs),
        grid=grid,
        in_specs=[
            pl.BlockSpec(memory_space=pltpu.SMEM),
            pl.BlockSpec((1, inner, pb), lambda i, j: (i, 0, j)),
            pl.BlockSpec((1, nc, pb), lambda i, j: (i, 0, j)),
            pl.BlockSpec((1, 1, pb), lambda i, j: (i, 0, j)),
            pl.BlockSpec((inner, nc * _QLEN), lambda i, j: (0, 0)),
        ],
        out_specs=pl.BlockSpec((1, 1), lambda i, j: (0, 0)),
        out_shape=jax.ShapeDtypeStruct((1, 1), jnp.float32),
        scratch_shapes=[
            pltpu.VMEM((inner, _NC_PAD), jnp.float32),
            pltpu.VMEM((1, _NC_PAD), jnp.float32),
            pltpu.VMEM((1, 1), jnp.float32),
        ],
        compiler_params=pltpu.CompilerParams(
            dimension_semantics=("arbitrary", "arbitrary"),
        ),
    )(lbs, fea_v, res_v, gt_v, q2d)
    return out[0, 0]


# speedup vs baseline: 5.1399x; 1.1661x over previous
import functools

import jax
import jax.numpy as jnp
from jax.experimental import pallas as pl
from jax.experimental.pallas import tpu as pltpu

_NUM_CLASSES = 5
_INNER = 256
_TEMP = 0.2
_QLEN = 64
_NC_PAD = 8


def _body(labelbs_ref, fea_ref, res_ref, gt_ref, q_ref, out_ref,
          acc_ref, cnt_ref, tot_ref, *, nblk, bs):
    ii = pl.program_id(0)
    j = pl.program_id(1)

    @pl.when(j == 0)
    def _init():
        acc_ref[...] = jnp.zeros_like(acc_ref)
        cnt_ref[...] = jnp.zeros_like(cnt_ref)

    @pl.when((j == 0) & (ii == 0))
    def _init_total():
        tot_ref[...] = jnp.zeros_like(tot_ref)

    fea = fea_ref[0]
    res = res_ref[0]
    gt_row = gt_ref[0]

    best_val = res[0:1, :]
    best_idx = jnp.zeros_like(gt_row)
    for c in range(1, _NUM_CLASSES):
        row = res[c:c + 1, :]
        upd = row > best_val
        best_val = jnp.where(upd, row, best_val)
        best_idx = jnp.where(upd, jnp.full_like(best_idx, c), best_idx)

    labels = jnp.where(ii < labelbs_ref[0, 0], gt_row, best_idx)

    cls_iota = jax.lax.broadcasted_iota(jnp.int32, (_NC_PAD, labels.shape[1]), 0)
    onehot = (cls_iota == labels).astype(jnp.float32)

    acc_ref[...] += jax.lax.dot_general(
        fea, onehot, (((1,), (1,)), ((), ())),
        preferred_element_type=jnp.float32)
    ones_row = jnp.ones((1, labels.shape[1]), jnp.float32)
    cnt_ref[...] += jax.lax.dot_general(
        ones_row, onehot, (((1,), (1,)), ((), ())),
        preferred_element_type=jnp.float32)

    @pl.when(j == nblk - 1)
    def _sample_loss():
        sums = acc_ref[:, :_NUM_CLASSES]
        counts = cnt_ref[:, :_NUM_CLASSES]
        means = sums / counts
        norm = jnp.sqrt(jnp.sum(means * means, axis=0, keepdims=True))
        keys = means / jnp.maximum(norm, 1e-12)
        logits = jax.lax.dot_general(
            keys, q_ref[...], (((0,), (0,)), ((), ())),
            preferred_element_type=jnp.float32)
        scaled = logits * (1.0 / _TEMP)
        expx = jnp.exp(scaled)
        loss_s = jnp.zeros((), jnp.float32)
        for cls in range(1, _NUM_CLASSES):
            row = scaled[cls - 1:cls, :]
            erow = expx[cls - 1:cls, :]
            l_pos = row[:, cls * _QLEN:(cls + 1) * _QLEN]
            e_pos = erow[:, cls * _QLEN:(cls + 1) * _QLEN]
            neg_base = jnp.sum(erow) - jnp.sum(e_pos)
            log_prob = l_pos - jnp.log(e_pos + neg_base)
            loss_s = loss_s + (-jnp.mean(log_prob))
        tot_ref[...] = tot_ref[...] + loss_s / (_NUM_CLASSES - 1)

    @pl.when((j == nblk - 1) & (ii == bs - 1))
    def _emit():
        out_ref[...] = tot_ref[...] / bs


@jax.jit
def kernel(res1, fea1, label_bs, gt, queues):
    bs, nc, h, w = res1.shape
    inner = fea1.shape[1]
    p = h * w
    pb = 16384
    nblk = p // pb

    fea_v = fea1.reshape(bs, inner, p)
    res_v = res1.reshape(bs, nc, p)
    gt_v = gt.reshape(bs, 1, p)
    q2d = jnp.transpose(queues, (1, 0, 2)).reshape(inner, nc * _QLEN)
    lbs = jnp.asarray(label_bs, jnp.int32).reshape(1, 1)

    grid = (bs, nblk)
    out = pl.pallas_call(
        functools.partial(_body, nblk=nblk, bs=bs),
        grid=grid,
        in_specs=[
            pl.BlockSpec(memory_space=pltpu.SMEM),
            pl.BlockSpec((1, inner, pb), lambda i, j: (i, 0, j)),
            pl.BlockSpec((1, nc, pb), lambda i, j: (i, 0, j)),
            pl.BlockSpec((1, 1, pb), lambda i, j: (i, 0, j)),
            pl.BlockSpec((inner, nc * _QLEN), lambda i, j: (0, 0)),
        ],
        out_specs=pl.BlockSpec((1, 1), lambda i, j: (0, 0)),
        out_shape=jax.ShapeDtypeStruct((1, 1), jnp.float32),
        scratch_shapes=[
            pltpu.VMEM((inner, _NC_PAD), jnp.float32),
            pltpu.VMEM((1, _NC_PAD), jnp.float32),
            pltpu.VMEM((1, 1), jnp.float32),
        ],
        compiler_params=pltpu.CompilerParams(
            dimension_semantics=("arbitrary", "arbitrary"),
        ),
    )(lbs, fea_v, res_v, gt_v, q2d)
    return out[0, 0]
